# baseline (speedup 1.0000x reference)
import jax
import jax.numpy as jnp
import numpy as np
from jax.experimental import pallas as pl
from jax.experimental.pallas import tpu as pltpu

B, T, H = 4, 2048, 2048
KC, KE = 64, 64
TOTAL = KC + KE
TOPK = 8
HIDDEN_PARAMS = 2000000
WIDTH = max(32, HIDDEN_PARAMS // (H + TOTAL))

TILE_T = 256
NT = T // TILE_T
PACK = 16
RP = H // PACK
INV_SQRT_H = 1.0 / np.sqrt(H)


def _body(temp_ref, hidden_ref, cls_ref, w1_ref, b1_ref, w2_ref, b2_ref,
          dc_ref, de_ref, out_ref, upd_ref):
    t = pl.program_id(0)

    @pl.when(t == 0)
    def _compute_update():
        cls = cls_ref[:, 0, :]
        cls3 = (cls.reshape(B, RP, PACK).swapaxes(1, 2)
                .reshape(B * PACK, RP))
        y = jnp.dot(cls3, w1_ref[...], preferred_element_type=jnp.float32)
        y3 = y.reshape(B, PACK, PACK * WIDTH)
        h1 = sum(y3[:, j, j * WIDTH:(j + 1) * WIDTH] for j in range(PACK))
        h1 = jnp.maximum(h1 + b1_ref[...], 0.0)
        logits = (jnp.dot(h1, w2_ref[...], preferred_element_type=jnp.float32)
                  + b2_ref[...]) / jnp.abs(temp_ref[0, 0])
        m = jnp.max(logits, axis=-1, keepdims=True)
        e = jnp.exp(logits - m)
        probs = e / jnp.sum(e, axis=-1, keepdims=True)
        iota = jax.lax.broadcasted_iota(jnp.int32, probs.shape, 1)
        remaining = probs
        coeff = jnp.zeros_like(probs)
        for _ in range(TOPK):
            cur = jnp.max(remaining, axis=-1, keepdims=True)
            ismax = remaining == cur
            first = jnp.min(jnp.where(ismax, iota, jnp.int32(2**30)),
                            axis=-1, keepdims=True)
            sel = iota == first
            coeff = jnp.where(sel, probs, coeff)
            remaining = jnp.where(sel, -jnp.inf, remaining)
        upd = (jnp.dot(coeff[:, :KC], dc_ref[...],
                       preferred_element_type=jnp.float32)
               + jnp.dot(coeff[:, KC:], de_ref[...],
                         preferred_element_type=jnp.float32))
        nrm = jnp.sqrt(jnp.sum(upd * upd, axis=-1, keepdims=True))
        upd_ref[...] = upd / jnp.maximum(nrm, 1e-12) * INV_SQRT_H

    out_ref[...] = hidden_ref[...] + upd_ref[...][:, None, :]


def kernel(hidden, D_c, D_e, W1, b1, W2, b2, temperature):
    temp = jnp.reshape(temperature, (1, 1))
    b1r = jnp.reshape(b1, (1, WIDTH))
    b2r = jnp.reshape(b2, (1, TOTAL))

    out = pl.pallas_call(
        _body,
        grid=(NT,),
        in_specs=[
            pl.BlockSpec(memory_space=pltpu.SMEM),
            pl.BlockSpec((B, TILE_T, H), lambda t: (0, t, 0)),
            pl.BlockSpec((B, 8, H), lambda t: (0, 0, 0)),
            pl.BlockSpec((RP, PACK * WIDTH), lambda t: (0, 0)),
            pl.BlockSpec((1, WIDTH), lambda t: (0, 0)),
            pl.BlockSpec((WIDTH, TOTAL), lambda t: (0, 0)),
            pl.BlockSpec((1, TOTAL), lambda t: (0, 0)),
            pl.BlockSpec((KC, H), lambda t: (0, 0)),
            pl.BlockSpec((KE, H), lambda t: (0, 0)),
        ],
        out_specs=pl.BlockSpec((B, TILE_T, H), lambda t: (0, t, 0)),
        out_shape=jax.ShapeDtypeStruct((B, T, H), jnp.float32),
        scratch_shapes=[pltpu.VMEM((B, H), jnp.float32)],
    )(temp, hidden, hidden, jnp.reshape(W1, (RP, PACK * WIDTH)),
      b1r, W2, b2r, D_c, D_e)
    return out


# speedup vs baseline: 1.4600x; 1.4600x over previous
import jax
import jax.numpy as jnp
import numpy as np
from jax.experimental import pallas as pl
from jax.experimental.pallas import tpu as pltpu

B, T, H = 4, 2048, 2048
KC, KE = 64, 64
TOTAL = KC + KE
TOPK = 8
HIDDEN_PARAMS = 2000000
WIDTH = max(32, HIDDEN_PARAMS // (H + TOTAL))

TILE_T = 256
NT = T // TILE_T
NW1 = 8
W1CH = H // NW1
INV_SQRT_H = 1.0 / np.sqrt(H)


def _body(temp_ref, hidden_ref, cls_ref, *rest):
    w1_refs = rest[:NW1]
    (b1_ref, w2_ref, b2_ref, dc_ref, de_ref, out_ref, upd_ref) = rest[NW1:]
    t = pl.program_id(0)

    @pl.when(t == 0)
    def _compute_update():
        cls = cls_ref[:, 0, :]
        h1 = b1_ref[...].astype(jnp.float32)
        h1 = h1 + sum(
            jnp.dot(cls[:, k * W1CH:(k + 1) * W1CH], w1_refs[k][...],
                    preferred_element_type=jnp.float32)
            for k in range(NW1))
        h1 = jnp.maximum(h1, 0.0)
        logits = (jnp.dot(h1, w2_ref[...], preferred_element_type=jnp.float32)
                  + b2_ref[...]) / jnp.abs(temp_ref[0, 0])
        m = jnp.max(logits, axis=-1, keepdims=True)
        e = jnp.exp(logits - m)
        probs = e / jnp.sum(e, axis=-1, keepdims=True)
        iota = jax.lax.broadcasted_iota(jnp.int32, probs.shape, 1)
        remaining = probs
        coeff = jnp.zeros_like(probs)
        for _ in range(TOPK):
            cur = jnp.max(remaining, axis=-1, keepdims=True)
            ismax = remaining == cur
            first = jnp.min(jnp.where(ismax, iota, jnp.int32(2**30)),
                            axis=-1, keepdims=True)
            sel = iota == first
            coeff = jnp.where(sel, probs, coeff)
            remaining = jnp.where(sel, -jnp.inf, remaining)
        upd = (jnp.dot(coeff[:, :KC], dc_ref[...],
                       preferred_element_type=jnp.float32)
               + jnp.dot(coeff[:, KC:], de_ref[...],
                         preferred_element_type=jnp.float32))
        nrm = jnp.sqrt(jnp.sum(upd * upd, axis=-1, keepdims=True))
        upd_ref[...] = upd / jnp.maximum(nrm, 1e-12) * INV_SQRT_H

    out_ref[...] = hidden_ref[...] + upd_ref[...][:, None, :]


def kernel(hidden, D_c, D_e, W1, b1, W2, b2, temperature):
    temp = jnp.reshape(temperature, (1, 1))
    b1r = jnp.reshape(b1, (1, WIDTH))
    b2r = jnp.reshape(b2, (1, TOTAL))

    w1_specs = [
        pl.BlockSpec((W1CH, WIDTH), lambda t, k=k: (k, 0)) for k in range(NW1)
    ]
    out = pl.pallas_call(
        _body,
        grid=(NT,),
        in_specs=[
            pl.BlockSpec(memory_space=pltpu.SMEM),
            pl.BlockSpec((B, TILE_T, H), lambda t: (0, t, 0)),
            pl.BlockSpec((B, 8, H), lambda t: (0, 0, 0)),
            *w1_specs,
            pl.BlockSpec((1, WIDTH), lambda t: (0, 0)),
            pl.BlockSpec((WIDTH, TOTAL), lambda t: (0, 0)),
            pl.BlockSpec((1, TOTAL), lambda t: (0, 0)),
            pl.BlockSpec((KC, H), lambda t: (0, 0)),
            pl.BlockSpec((KE, H), lambda t: (0, 0)),
        ],
        out_specs=pl.BlockSpec((B, TILE_T, H), lambda t: (0, t, 0)),
        out_shape=jax.ShapeDtypeStruct((B, T, H), jnp.float32),
        scratch_shapes=[pltpu.VMEM((B, H), jnp.float32)],
    )(temp, hidden, hidden, *([W1] * NW1), b1r, W2, b2r, D_c, D_e)
    return out
